# initial kernel scaffold (unmeasured)
import jax
import jax.numpy as jnp
from jax import lax
from jax.experimental import pallas as pl
from jax.experimental.pallas import tpu as pltpu


def kernel(
    x,
):
    def body(*refs):
        pass

    out_shape = jax.ShapeDtypeStruct(..., jnp.float32)
    return pl.pallas_call(body, out_shape=out_shape)(...)


# baseline (device time: 1611443 ns/iter reference)
import jax
import jax.numpy as jnp
from jax import lax
from jax.experimental import pallas as pl
from jax.experimental.pallas import tpu as pltpu

CHUNK_ROWS = 2048


def kernel(x):
    m, n = x.shape
    half = m // 2

    def body(x_hbm, out_hbm, a_vmem, b_vmem, send_sems, recv_sems,
             local_sems, credit_sem):
        my_x = lax.axis_index("x")
        my_y = lax.axis_index("y")
        x_nbr = (1 - my_x, my_y)
        y_nbr = (my_x, 1 - my_y)

        barrier = pltpu.get_barrier_semaphore()
        for nbr in (x_nbr, y_nbr):
            pl.semaphore_signal(
                barrier, inc=1, device_id=nbr,
                device_id_type=pl.DeviceIdType.MESH,
            )
        pl.semaphore_wait(barrier, 2)

        off = my_y * half
        stage = (1 - my_y) * half

        rdma1 = pltpu.make_async_remote_copy(
            src_ref=x_hbm.at[pl.ds(off, half)],
            dst_ref=out_hbm.at[pl.ds(stage, half)],
            send_sem=send_sems.at[0],
            recv_sem=recv_sems.at[0],
            device_id=x_nbr,
            device_id_type=pl.DeviceIdType.MESH,
        )
        rdma1.start()
        rdma1.wait()

        def add_chunk(i, _):
            r0 = off + i * CHUNK_ROWS
            c0 = pltpu.make_async_copy(
                x_hbm.at[pl.ds(r0, CHUNK_ROWS)], a_vmem, local_sems.at[0])
            c1 = pltpu.make_async_copy(
                out_hbm.at[pl.ds(stage + i * CHUNK_ROWS, CHUNK_ROWS)], b_vmem,
                local_sems.at[1])
            c0.start()
            c1.start()
            c0.wait()
            c1.wait()
            a_vmem[...] = a_vmem[...] + b_vmem[...]
            c2 = pltpu.make_async_copy(
                a_vmem, out_hbm.at[pl.ds(r0, CHUNK_ROWS)], local_sems.at[2])
            c2.start()
            c2.wait()
            return 0

        lax.fori_loop(0, half // CHUNK_ROWS, add_chunk, 0)

        pl.semaphore_signal(
            credit_sem, inc=1, device_id=y_nbr,
            device_id_type=pl.DeviceIdType.MESH,
        )
        pl.semaphore_wait(credit_sem, 1)

        rdma2 = pltpu.make_async_remote_copy(
            src_ref=out_hbm.at[pl.ds(off, half)],
            dst_ref=out_hbm.at[pl.ds(off, half)],
            send_sem=send_sems.at[1],
            recv_sem=recv_sems.at[1],
            device_id=y_nbr,
            device_id_type=pl.DeviceIdType.MESH,
        )
        rdma2.start()
        rdma2.wait()

    return pl.pallas_call(
        body,
        out_shape=jax.ShapeDtypeStruct((m, n), x.dtype),
        in_specs=[pl.BlockSpec(memory_space=pl.ANY)],
        out_specs=pl.BlockSpec(memory_space=pl.ANY),
        scratch_shapes=[
            pltpu.VMEM((CHUNK_ROWS, n), x.dtype),
            pltpu.VMEM((CHUNK_ROWS, n), x.dtype),
            pltpu.SemaphoreType.DMA((2,)),
            pltpu.SemaphoreType.DMA((2,)),
            pltpu.SemaphoreType.DMA((3,)),
            pltpu.SemaphoreType.REGULAR,
        ],
        compiler_params=pltpu.CompilerParams(collective_id=0),
    )(x)


# device time: 864230 ns/iter; 1.8646x vs baseline; 1.8646x over previous
import jax
import jax.numpy as jnp
from jax import lax
from jax.experimental import pallas as pl
from jax.experimental.pallas import tpu as pltpu

NCHUNK = 16
MESH = pl.DeviceIdType.MESH


def kernel(x):
    m, n = x.shape
    half = m // 2
    cr = half // NCHUNK

    def body(x_hbm, out_hbm, a_vmem, b_vmem, send1, recv1, send2, recv2,
             local_sems, credit_sem):
        my_x = lax.axis_index("x")
        my_y = lax.axis_index("y")
        x_nbr = (1 - my_x, my_y)
        y_nbr = (my_x, 1 - my_y)

        barrier = pltpu.get_barrier_semaphore()
        for nbr in (x_nbr, y_nbr):
            pl.semaphore_signal(barrier, inc=1, device_id=nbr,
                                device_id_type=MESH)
        pl.semaphore_wait(barrier, 2)

        off = my_y * half
        stage = (1 - my_y) * half

        rdma1 = []
        for i in range(NCHUNK):
            r = pltpu.make_async_remote_copy(
                src_ref=x_hbm.at[pl.ds(off + i * cr, cr)],
                dst_ref=out_hbm.at[pl.ds(stage + i * cr, cr)],
                send_sem=send1.at[i],
                recv_sem=recv1.at[i],
                device_id=x_nbr,
                device_id_type=MESH,
            )
            r.start()
            rdma1.append(r)

        rdma2 = []
        for i in range(NCHUNK):
            rdma1[i].wait_recv()
            c0 = pltpu.make_async_copy(
                x_hbm.at[pl.ds(off + i * cr, cr)], a_vmem, local_sems.at[0])
            c1 = pltpu.make_async_copy(
                out_hbm.at[pl.ds(stage + i * cr, cr)], b_vmem,
                local_sems.at[1])
            c0.start()
            c1.start()
            c0.wait()
            c1.wait()
            pl.semaphore_signal(credit_sem, inc=1, device_id=y_nbr,
                                device_id_type=MESH)
            a_vmem[...] = a_vmem[...] + b_vmem[...]
            c2 = pltpu.make_async_copy(
                a_vmem, out_hbm.at[pl.ds(off + i * cr, cr)], local_sems.at[2])
            c2.start()
            c2.wait()
            pl.semaphore_wait(credit_sem, 1)
            r2 = pltpu.make_async_remote_copy(
                src_ref=out_hbm.at[pl.ds(off + i * cr, cr)],
                dst_ref=out_hbm.at[pl.ds(off + i * cr, cr)],
                send_sem=send2.at[i],
                recv_sem=recv2.at[i],
                device_id=y_nbr,
                device_id_type=MESH,
            )
            r2.start()
            rdma2.append(r2)

        for i in range(NCHUNK):
            rdma1[i].wait_send()
            rdma2[i].wait()

    return pl.pallas_call(
        body,
        out_shape=jax.ShapeDtypeStruct((m, n), x.dtype),
        in_specs=[pl.BlockSpec(memory_space=pl.ANY)],
        out_specs=pl.BlockSpec(memory_space=pl.ANY),
        scratch_shapes=[
            pltpu.VMEM((cr, n), x.dtype),
            pltpu.VMEM((cr, n), x.dtype),
            pltpu.SemaphoreType.DMA((NCHUNK,)),
            pltpu.SemaphoreType.DMA((NCHUNK,)),
            pltpu.SemaphoreType.DMA((NCHUNK,)),
            pltpu.SemaphoreType.DMA((NCHUNK,)),
            pltpu.SemaphoreType.DMA((3,)),
            pltpu.SemaphoreType.REGULAR,
        ],
        compiler_params=pltpu.CompilerParams(collective_id=0),
    )(x)


# device time: 839095 ns/iter; 1.9205x vs baseline; 1.0300x over previous
import jax
import jax.numpy as jnp
from jax import lax
from jax.experimental import pallas as pl
from jax.experimental.pallas import tpu as pltpu

NCHUNK = 32
MESH = pl.DeviceIdType.MESH


def kernel(x):
    m, n = x.shape
    half = m // 2
    cr = half // NCHUNK

    def body(x_hbm, out_hbm, a_vmem, b_vmem, send1, recv1, send2, recv2,
             in_sems, out_sems, credit_sem):
        my_x = lax.axis_index("x")
        my_y = lax.axis_index("y")
        x_nbr = (1 - my_x, my_y)
        y_nbr = (my_x, 1 - my_y)

        barrier = pltpu.get_barrier_semaphore()
        for nbr in (x_nbr, y_nbr):
            pl.semaphore_signal(barrier, inc=1, device_id=nbr,
                                device_id_type=MESH)
        pl.semaphore_wait(barrier, 2)

        off = my_y * half
        stage = (1 - my_y) * half

        rdma1 = []
        for i in range(NCHUNK):
            r = pltpu.make_async_remote_copy(
                src_ref=x_hbm.at[pl.ds(off + i * cr, cr)],
                dst_ref=out_hbm.at[pl.ds(stage + i * cr, cr)],
                send_sem=send1.at[i],
                recv_sem=recv1.at[i],
                device_id=x_nbr,
                device_id_type=MESH,
            )
            r.start()
            rdma1.append(r)

        rdma2 = []
        c2s = []
        for i in range(NCHUNK):
            s = i % 2
            if i >= 2:
                rdma2[i - 2].wait_send()
                c2s[i - 2].wait()
            c0 = pltpu.make_async_copy(
                x_hbm.at[pl.ds(off + i * cr, cr)], a_vmem.at[s],
                in_sems.at[s])
            c0.start()
            rdma1[i].wait_recv()
            c1 = pltpu.make_async_copy(
                out_hbm.at[pl.ds(stage + i * cr, cr)], b_vmem.at[s],
                in_sems.at[2 + s])
            c1.start()
            c0.wait()
            c1.wait()
            pl.semaphore_signal(credit_sem, inc=1, device_id=y_nbr,
                                device_id_type=MESH)
            a_vmem[s] = a_vmem[s] + b_vmem[s]
            c2 = pltpu.make_async_copy(
                a_vmem.at[s], out_hbm.at[pl.ds(off + i * cr, cr)],
                out_sems.at[s])
            c2.start()
            c2s.append(c2)
            pl.semaphore_wait(credit_sem, 1)
            r2 = pltpu.make_async_remote_copy(
                src_ref=a_vmem.at[s],
                dst_ref=out_hbm.at[pl.ds(off + i * cr, cr)],
                send_sem=send2.at[i],
                recv_sem=recv2.at[i],
                device_id=y_nbr,
                device_id_type=MESH,
            )
            r2.start()
            rdma2.append(r2)

        for i in range(NCHUNK):
            rdma1[i].wait_send()
            rdma2[i].wait_recv()
        for i in (NCHUNK - 2, NCHUNK - 1):
            rdma2[i].wait_send()
            c2s[i].wait()

    return pl.pallas_call(
        body,
        out_shape=jax.ShapeDtypeStruct((m, n), x.dtype),
        in_specs=[pl.BlockSpec(memory_space=pl.ANY)],
        out_specs=pl.BlockSpec(memory_space=pl.ANY),
        scratch_shapes=[
            pltpu.VMEM((2, cr, n), x.dtype),
            pltpu.VMEM((2, cr, n), x.dtype),
            pltpu.SemaphoreType.DMA((NCHUNK,)),
            pltpu.SemaphoreType.DMA((NCHUNK,)),
            pltpu.SemaphoreType.DMA((NCHUNK,)),
            pltpu.SemaphoreType.DMA((NCHUNK,)),
            pltpu.SemaphoreType.DMA((4,)),
            pltpu.SemaphoreType.DMA((2,)),
            pltpu.SemaphoreType.REGULAR,
        ],
        compiler_params=pltpu.CompilerParams(collective_id=0),
    )(x)
